# Initial kernel scaffold; baseline (speedup 1.0000x reference)
import jax
import jax.numpy as jnp
from jax.experimental import pallas as pl


def kernel(seq1, adj, sparse, W, bias, prelu_a):
    raise NotImplementedError("write your pallas kernel here")


# speedup vs baseline: 1.0427x; 1.0427x over previous
import functools

import jax
import jax.numpy as jnp
from jax.experimental import pallas as pl
from jax.experimental.pallas import tpu as pltpu

N = 10000
D_IN = 128
D_H = 128
BLOCK_M = 400


def _gcn_kernel(x_ref, w_ref, a_ref, b_ref, p_ref, o_ref, fts_ref):
    @pl.when(pl.program_id(0) == 0)
    def _():
        fts_ref[...] = jax.lax.dot_general(
            x_ref[...], w_ref[...],
            dimension_numbers=(((1,), (1,)), ((), ())),
            preferred_element_type=jnp.float32)

    acc = jnp.dot(a_ref[...], fts_ref[...], preferred_element_type=jnp.float32)
    acc = acc + b_ref[...]
    slope = p_ref[0, 0]
    o_ref[...] = jnp.where(acc >= 0.0, acc, slope * acc)


@functools.partial(jax.jit, static_argnames=())
def _gcn_forward(x, w, a, b, p):
    grid = (N // BLOCK_M,)
    return pl.pallas_call(
        _gcn_kernel,
        grid=grid,
        in_specs=[
            pl.BlockSpec((N, D_IN), lambda i: (0, 0)),
            pl.BlockSpec((D_H, D_IN), lambda i: (0, 0)),
            pl.BlockSpec((BLOCK_M, N), lambda i: (i, 0)),
            pl.BlockSpec((1, D_H), lambda i: (0, 0)),
            pl.BlockSpec((1, 1), lambda i: (0, 0)),
        ],
        out_specs=pl.BlockSpec((BLOCK_M, D_H), lambda i: (i, 0)),
        out_shape=jax.ShapeDtypeStruct((N, D_H), jnp.float32),
        scratch_shapes=[pltpu.VMEM((N, D_H), jnp.float32)],
    )(x, w, a, b, p)


def kernel(seq1, adj, sparse, W, bias, prelu_a):
    del sparse
    x = seq1[0]
    a = adj[0]
    b = bias.reshape(1, D_H)
    p = prelu_a.reshape(1, 1)
    out = _gcn_forward(x, W, a, b, p)
    return out[None]
